# baseline (speedup 1.0000x reference)
import functools

import jax
import jax.numpy as jnp
from jax import lax
from jax.experimental import pallas as pl
from jax.experimental.pallas import tpu as pltpu
from jax.experimental.pallas import tpu_sc as plsc

B, T, D, Dc, K = 8, 1024, 768, 64, 8192
N = B * T
NCH = 1
H = N // NCH


TM1 = 256
GH1 = H // TM1


def _vq_body(x_ref, wi_ref, bi_ref, cbt_ref, z_ref, idx_ref, e2_ref):
    @pl.when(pl.program_id(0) == 0)
    def _():
        cbt = cbt_ref[...]
        e2 = jnp.sum(cbt * cbt, axis=0, keepdims=True)
        row = jax.lax.broadcasted_iota(jnp.int32, (8, K), 0)
        e2_ref[...] = jnp.where(row == 0, jnp.broadcast_to(e2, (8, K)), 0.0)

    x = x_ref[...]
    z = jnp.dot(x, wi_ref[...], preferred_element_type=jnp.float32) + bi_ref[...]
    z_ref[...] = z
    s = jnp.dot(-2.0 * z, cbt_ref[...], preferred_element_type=jnp.float32)
    d = s + e2_ref[0:1, :]
    idx = jnp.argmin(d, axis=1).astype(jnp.int32)
    idx_ref[...] = idx.reshape(1, 1, TM1)


def _project_quantize(xh, W_in, b_in2, cbT):
    return pl.pallas_call(
        _vq_body,
        grid=(GH1,),
        in_specs=[
            pl.BlockSpec((TM1, D), lambda i: (i, 0)),
            pl.BlockSpec((D, Dc), lambda i: (0, 0)),
            pl.BlockSpec((1, Dc), lambda i: (0, 0)),
            pl.BlockSpec((Dc, K), lambda i: (0, 0)),
        ],
        out_specs=[
            pl.BlockSpec((TM1, Dc), lambda i: (i, 0)),
            pl.BlockSpec((1, 1, TM1), lambda i: (i, 0, 0)),
        ],
        out_shape=[
            jax.ShapeDtypeStruct((H, Dc), jnp.float32),
            jax.ShapeDtypeStruct((GH1, 1, TM1), jnp.int32),
        ],
        scratch_shapes=[pltpu.VMEM((8, K), jnp.float32)],
    )(xh, W_in, b_in2, cbT)


NC, NS, L = 2, 16, 16
NW = NC * NS
BPW = H // NW
DP = 128


def _gather_body(cb_hbm, idx_hbm, out_hbm, idx_v, rows_v, sem):
    wid = lax.axis_index("s") * NC + lax.axis_index("c")
    base = wid * BPW
    pltpu.sync_copy(idx_hbm.at[pl.ds(base, BPW)], idx_v)
    pltpu.async_copy(cb_hbm.at[idx_v], rows_v, sem).wait()
    pltpu.sync_copy(rows_v, out_hbm.at[pl.ds(base, BPW)])


@functools.lru_cache(maxsize=1)
def _codebook_gather():
    return pl.kernel(
        _gather_body,
        mesh=plsc.VectorSubcoreMesh(core_axis_name="c", subcore_axis_name="s"),
        out_type=jax.ShapeDtypeStruct((H, DP), jnp.float32),
        scratch_types=[
            pltpu.VMEM((BPW,), jnp.int32),
            pltpu.VMEM((BPW, DP), jnp.float32),
            pltpu.SemaphoreType.DMA,
        ],
    )


TM2 = 512
GH2 = H // TM2


def _out_body(q_ref, z_ref, wo_ref, bo_ref, *rest):
    if len(rest) == 3:
        _, qf_ref, acc_ref = rest
    else:
        qf_ref, acc_ref = rest
    q = q_ref[:, :Dc]
    z = z_ref[...]
    diff = q - z
    q_st = z + diff
    qf_ref[...] = jnp.dot(q_st, wo_ref[...], preferred_element_type=jnp.float32) + bo_ref[...]

    @pl.when(pl.program_id(0) == 0)
    def _():
        acc_ref[0, 0] = 0.0

    acc_ref[0, 0] += jnp.sum(diff * diff)


def _project_out(chunk, q, z, W_out, b_out2, qf_prev=None):
    blk = chunk * GH2
    in_specs = [
        pl.BlockSpec((TM2, DP), lambda i: (i, 0)),
        pl.BlockSpec((TM2, Dc), lambda i: (i, 0)),
        pl.BlockSpec((Dc, D), lambda i: (0, 0)),
        pl.BlockSpec((1, D), lambda i: (0, 0)),
    ]
    args = [q, z, W_out, b_out2]
    aliases = {}
    if qf_prev is not None:
        in_specs.append(pl.BlockSpec(memory_space=pl.ANY))
        args.append(qf_prev)
        aliases = {4: 0}
    return pl.pallas_call(
        _out_body,
        grid=(GH2,),
        in_specs=in_specs,
        out_specs=[
            pl.BlockSpec((TM2, D), lambda i: (i + blk, 0)),
            pl.BlockSpec(memory_space=pltpu.SMEM),
        ],
        out_shape=[
            jax.ShapeDtypeStruct((N, D), jnp.float32),
            jax.ShapeDtypeStruct((1, 1), jnp.float32),
        ],
        input_output_aliases=aliases,
    )(*args)


def kernel(x, W_in, b_in, codebook, W_out, b_out):
    xr = x.reshape(N, D)
    cbT = codebook.T
    b_in2 = b_in.reshape(1, Dc)
    b_out2 = b_out.reshape(1, D)
    cb_pad = jnp.pad(codebook, ((0, 0), (0, DP - Dc)))

    zs, idxs, qs = [], [], []
    for c in range(NCH):
        z, idx3 = _project_quantize(xr[c * H:(c + 1) * H], W_in, b_in2, cbT)
        zs.append(z)
        idxs.append(idx3.reshape(H))
        qs.append(_codebook_gather()(cb_pad, idxs[c]))

    qf = None
    accs = []
    for c in range(NCH):
        qf, acc = _project_out(c, qs[c], zs[c], W_out, b_out2, qf)
        accs.append(acc[0, 0])

    vq_loss = sum(accs) * (1.25 / (N * Dc))
    indices = jnp.concatenate(idxs)
    return (qf.reshape(B, T, D), indices.reshape(B, T), vq_loss)


# speedup vs baseline: 1.0685x; 1.0685x over previous
import functools

import jax
import jax.numpy as jnp
from jax import lax
from jax.experimental import pallas as pl
from jax.experimental.pallas import tpu as pltpu
from jax.experimental.pallas import tpu_sc as plsc

B, T, D, Dc, K = 8, 1024, 768, 64, 8192
N = B * T
NCH = 1
H = N // NCH


TM1 = 1024
GH1 = H // TM1


def _vq_body(x_ref, wi_ref, bi_ref, cbt_ref, z_ref, idx_ref, e2_ref):
    @pl.when(pl.program_id(0) == 0)
    def _():
        cbt = cbt_ref[...]
        e2 = jnp.sum(cbt * cbt, axis=0, keepdims=True)
        row = jax.lax.broadcasted_iota(jnp.int32, (8, K), 0)
        e2_ref[...] = jnp.where(row == 0, jnp.broadcast_to(e2, (8, K)), 0.0)

    x = x_ref[...]
    z = jnp.dot(x, wi_ref[...], preferred_element_type=jnp.float32) + bi_ref[...]
    z_ref[...] = z
    s = jnp.dot(-2.0 * z, cbt_ref[...], preferred_element_type=jnp.float32)
    d = s + e2_ref[0:1, :]
    idx = jnp.argmin(d, axis=1).astype(jnp.int32)
    idx_ref[...] = idx.reshape(1, 1, TM1)


def _project_quantize(xh, W_in, b_in2, cbT):
    return pl.pallas_call(
        _vq_body,
        grid=(GH1,),
        in_specs=[
            pl.BlockSpec((TM1, D), lambda i: (i, 0)),
            pl.BlockSpec((D, Dc), lambda i: (0, 0)),
            pl.BlockSpec((1, Dc), lambda i: (0, 0)),
            pl.BlockSpec((Dc, K), lambda i: (0, 0)),
        ],
        out_specs=[
            pl.BlockSpec((TM1, Dc), lambda i: (i, 0)),
            pl.BlockSpec((1, 1, TM1), lambda i: (i, 0, 0)),
        ],
        out_shape=[
            jax.ShapeDtypeStruct((H, Dc), jnp.float32),
            jax.ShapeDtypeStruct((GH1, 1, TM1), jnp.int32),
        ],
        scratch_shapes=[pltpu.VMEM((8, K), jnp.float32)],
    )(xh, W_in, b_in2, cbT)


NC, NS, L = 2, 16, 16
NW = NC * NS
BPW = H // NW
DP = 128


def _gather_body(cb_hbm, idx_hbm, out_hbm, idx_v, rows_v, sem):
    wid = lax.axis_index("s") * NC + lax.axis_index("c")
    base = wid * BPW
    pltpu.sync_copy(idx_hbm.at[pl.ds(base, BPW)], idx_v)
    pltpu.async_copy(cb_hbm.at[idx_v], rows_v, sem).wait()
    pltpu.sync_copy(rows_v, out_hbm.at[pl.ds(base, BPW)])


@functools.lru_cache(maxsize=1)
def _codebook_gather():
    return pl.kernel(
        _gather_body,
        mesh=plsc.VectorSubcoreMesh(core_axis_name="c", subcore_axis_name="s"),
        out_type=jax.ShapeDtypeStruct((H, DP), jnp.float32),
        scratch_types=[
            pltpu.VMEM((BPW,), jnp.int32),
            pltpu.VMEM((BPW, DP), jnp.float32),
            pltpu.SemaphoreType.DMA,
        ],
    )


TM2 = 512
GH2 = H // TM2


def _out_body(q_ref, z_ref, wo_ref, bo_ref, *rest):
    if len(rest) == 3:
        _, qf_ref, acc_ref = rest
    else:
        qf_ref, acc_ref = rest
    q = q_ref[:, :Dc]
    z = z_ref[...]
    diff = q - z
    q_st = z + diff
    qf_ref[...] = jnp.dot(q_st, wo_ref[...], preferred_element_type=jnp.float32) + bo_ref[...]

    @pl.when(pl.program_id(0) == 0)
    def _():
        acc_ref[0, 0] = 0.0

    acc_ref[0, 0] += jnp.sum(diff * diff)


def _project_out(chunk, q, z, W_out, b_out2, qf_prev=None):
    blk = chunk * GH2
    in_specs = [
        pl.BlockSpec((TM2, DP), lambda i: (i, 0)),
        pl.BlockSpec((TM2, Dc), lambda i: (i, 0)),
        pl.BlockSpec((Dc, D), lambda i: (0, 0)),
        pl.BlockSpec((1, D), lambda i: (0, 0)),
    ]
    args = [q, z, W_out, b_out2]
    aliases = {}
    if qf_prev is not None:
        in_specs.append(pl.BlockSpec(memory_space=pl.ANY))
        args.append(qf_prev)
        aliases = {4: 0}
    return pl.pallas_call(
        _out_body,
        grid=(GH2,),
        in_specs=in_specs,
        out_specs=[
            pl.BlockSpec((TM2, D), lambda i: (i + blk, 0)),
            pl.BlockSpec(memory_space=pltpu.SMEM),
        ],
        out_shape=[
            jax.ShapeDtypeStruct((N, D), jnp.float32),
            jax.ShapeDtypeStruct((1, 1), jnp.float32),
        ],
        input_output_aliases=aliases,
    )(*args)


def kernel(x, W_in, b_in, codebook, W_out, b_out):
    xr = x.reshape(N, D)
    cbT = codebook.T
    b_in2 = b_in.reshape(1, Dc)
    b_out2 = b_out.reshape(1, D)
    cb_pad = jnp.pad(codebook, ((0, 0), (0, DP - Dc)))

    zs, idxs, qs = [], [], []
    for c in range(NCH):
        z, idx3 = _project_quantize(xr[c * H:(c + 1) * H], W_in, b_in2, cbT)
        zs.append(z)
        idxs.append(idx3.reshape(H))
        qs.append(_codebook_gather()(cb_pad, idxs[c]))

    qf = None
    accs = []
    for c in range(NCH):
        qf, acc = _project_out(c, qs[c], zs[c], W_out, b_out2, qf)
        accs.append(acc[0, 0])

    vq_loss = sum(accs) * (1.25 / (N * Dc))
    indices = jnp.concatenate(idxs)
    return (qf.reshape(B, T, D), indices.reshape(B, T), vq_loss)
